# baseline (speedup 1.0000x reference)
import functools

import jax
import jax.numpy as jnp
from jax import lax
from jax.experimental import pallas as pl
from jax.experimental.pallas import tpu as pltpu
from jax.experimental.pallas import tpu_sc as plsc


def _node_scores_body(block, u_ref, i_ref, w_ref, b_ref, us_ref, is_ref):
    g = pl.program_id(0)
    d_u = u_ref.shape[1]
    wu = w_ref[:, :d_u]
    wi = w_ref[:, d_u:]
    dn = (((1,), (1,)), ((), ()))
    us = lax.dot_general(wu, u_ref[...], dn, preferred_element_type=jnp.float32)
    vs = lax.dot_general(wi, i_ref[...], dn, preferred_element_type=jnp.float32)
    sl = pl.ds(g * block, block)
    us_ref[sl] = us[0]
    is_ref[sl] = vs[0] + b_ref[0, 0]


def _node_scores(user_feat, item_feat, W, b):
    n_user, d_u = user_feat.shape
    n_item, d_i = item_feat.shape
    block = 2048
    grid = ((n_user + block - 1) // block,)
    n_pad = grid[0] * block
    b2 = b.reshape(1, 1)
    return pl.pallas_call(
        functools.partial(_node_scores_body, block),
        grid=grid,
        in_specs=[
            pl.BlockSpec((block, d_u), lambda i: (i, 0)),
            pl.BlockSpec((block, d_i), lambda i: (i, 0)),
            pl.BlockSpec((1, d_u + d_i), lambda i: (0, 0)),
            pl.BlockSpec((1, 1), lambda i: (0, 0)),
        ],
        out_specs=[
            pl.BlockSpec((n_pad,), lambda i: (0,)),
            pl.BlockSpec((n_pad,), lambda i: (0,)),
        ],
        out_shape=[
            jax.ShapeDtypeStruct((n_pad,), jnp.float32),
            jax.ShapeDtypeStruct((n_pad,), jnp.float32),
        ],
    )(user_feat, item_feat, W, b2)


_N_CHUNKS = 5


def _edge_scores(u_score, i_score, edge_src, edge_dst):
    n_user = u_score.shape[0]
    n_item = i_score.shape[0]
    e = edge_src.shape[0]
    info = plsc.get_sparse_core_info()
    nc, ns = info.num_cores, info.num_subcores
    nw = nc * ns
    epw = e // nw
    mesh = plsc.VectorSubcoreMesh(core_axis_name="c", subcore_axis_name="s")

    @functools.partial(
        pl.kernel,
        mesh=mesh,
        compiler_params=pltpu.CompilerParams(
            needs_layout_passes=False, use_tc_tiling_on_sc=False),
        out_type=jax.ShapeDtypeStruct((e,), jnp.float32),
        scratch_types=[
            pltpu.VMEM((epw,), jnp.int32),
            pltpu.VMEM((epw,), jnp.int32),
            pltpu.VMEM((n_user,), jnp.float32),
            pltpu.VMEM((n_item,), jnp.float32),
            pltpu.VMEM((epw,), jnp.float32),
            pltpu.SemaphoreType.DMA,
            pltpu.SemaphoreType.DMA,
            [pltpu.SemaphoreType.DMA] * _N_CHUNKS,
        ],
    )
    def sc_kernel(us_hbm, is_hbm, src_hbm, dst_hbm, out_hbm,
                  src_v, dst_v, us_v, is_v, out_v,
                  tab_sem, out_sem, in_sems):
        wid = lax.axis_index("s") * nc + lax.axis_index("c")
        base = wid * epw
        ce = epw // _N_CHUNKS
        citer = ce // 16
        t1 = pltpu.async_copy(us_hbm, us_v, tab_sem)
        t2 = pltpu.async_copy(is_hbm, is_v, tab_sem)
        in_copies = []
        for k in range(_N_CHUNKS):
            o = k * ce
            in_copies.append((
                pltpu.async_copy(src_hbm.at[pl.ds(base + o, ce)],
                                 src_v.at[pl.ds(o, ce)], in_sems[k]),
                pltpu.async_copy(dst_hbm.at[pl.ds(base + o, ce)],
                                 dst_v.at[pl.ds(o, ce)], in_sems[k]),
            ))
        t1.wait()
        t2.wait()
        out_copies = []
        for k in range(_N_CHUNKS):
            c1, c2 = in_copies[k]
            c1.wait()
            c2.wait()

            @plsc.parallel_loop(k * citer, (k + 1) * citer, unroll=25)
            def body(i):
                sl = pl.ds(i * 16, 16)
                vu = plsc.load_gather(us_v, [src_v[sl]])
                vi = plsc.load_gather(is_v, [dst_v[sl]])
                out_v[sl] = vu + vi

            o = k * ce
            out_copies.append(
                pltpu.async_copy(out_v.at[pl.ds(o, ce)],
                                 out_hbm.at[pl.ds(base + o, ce)], out_sem))
        for c in out_copies:
            c.wait()

    return sc_kernel(u_score, i_score, edge_src, edge_dst)


def kernel(user_feat, item_feat, edge_src, edge_dst, W, b):
    u_score, i_score = _node_scores(user_feat, item_feat, W, b)
    out = _edge_scores(u_score, i_score, edge_src, edge_dst)
    return out.reshape(edge_src.shape[0], 1)


# speedup vs baseline: 1.1025x; 1.1025x over previous
import functools

import jax
import jax.numpy as jnp
from jax import lax
from jax.experimental import pallas as pl
from jax.experimental.pallas import tpu as pltpu
from jax.experimental.pallas import tpu_sc as plsc


def _node_scores_body(block, u_ref, i_ref, w_ref, b_ref, us_ref, is_ref):
    g = pl.program_id(0)
    d_u = u_ref.shape[1]
    wu = w_ref[:, :d_u]
    wi = w_ref[:, d_u:]
    dn = (((1,), (1,)), ((), ()))
    us = lax.dot_general(wu, u_ref[...], dn, preferred_element_type=jnp.float32)
    vs = lax.dot_general(wi, i_ref[...], dn, preferred_element_type=jnp.float32)
    sl = pl.ds(g * block, block)
    us_ref[sl] = us[0]
    is_ref[sl] = vs[0] + b_ref[0, 0]


def _node_scores(user_feat, item_feat, W, b):
    n_user, d_u = user_feat.shape
    n_item, d_i = item_feat.shape
    block = 2048
    grid = ((n_user + block - 1) // block,)
    n_pad = grid[0] * block
    b2 = b.reshape(1, 1)
    return pl.pallas_call(
        functools.partial(_node_scores_body, block),
        grid=grid,
        in_specs=[
            pl.BlockSpec((block, d_u), lambda i: (i, 0)),
            pl.BlockSpec((block, d_i), lambda i: (i, 0)),
            pl.BlockSpec((1, d_u + d_i), lambda i: (0, 0)),
            pl.BlockSpec((1, 1), lambda i: (0, 0)),
        ],
        out_specs=[
            pl.BlockSpec((n_pad,), lambda i: (0,)),
            pl.BlockSpec((n_pad,), lambda i: (0,)),
        ],
        out_shape=[
            jax.ShapeDtypeStruct((n_pad,), jnp.float32),
            jax.ShapeDtypeStruct((n_pad,), jnp.float32),
        ],
    )(user_feat, item_feat, W, b2)


_N_CHUNKS = 5


def _edge_scores(u_score, i_score, edge_src, edge_dst):
    n_user = u_score.shape[0]
    n_item = i_score.shape[0]
    e = edge_src.shape[0]
    info = plsc.get_sparse_core_info()
    nc, ns = info.num_cores, info.num_subcores
    nw = nc * ns
    epw = e // nw
    mesh = plsc.VectorSubcoreMesh(core_axis_name="c", subcore_axis_name="s")

    @functools.partial(
        pl.kernel,
        mesh=mesh,
        compiler_params=pltpu.CompilerParams(
            needs_layout_passes=False, use_tc_tiling_on_sc=False),
        out_type=jax.ShapeDtypeStruct((e,), jnp.float32),
        scratch_types=[
            pltpu.VMEM((epw,), jnp.int32),
            pltpu.VMEM((epw,), jnp.int32),
            pltpu.VMEM((n_user,), jnp.float32),
            pltpu.VMEM((n_item,), jnp.float32),
            pltpu.VMEM((epw,), jnp.float32),
            pltpu.VMEM_SHARED((n_user,), jnp.float32),
            pltpu.VMEM_SHARED((n_item,), jnp.float32),
            pltpu.SemaphoreType.DMA,
            pltpu.SemaphoreType.DMA,
            [pltpu.SemaphoreType.DMA] * _N_CHUNKS,
        ],
    )
    def sc_kernel(us_hbm, is_hbm, src_hbm, dst_hbm, out_hbm,
                  src_v, dst_v, us_v, is_v, out_v, us_sh, is_sh,
                  tab_sem, out_sem, in_sems):
        sid = lax.axis_index("s")
        wid = sid * nc + lax.axis_index("c")
        base = wid * epw
        ce = epw // _N_CHUNKS
        citer = ce // 16

        @pl.when(sid == 0)
        def _():
            s1 = pltpu.async_copy(us_hbm, us_sh, tab_sem)
            s2 = pltpu.async_copy(is_hbm, is_sh, tab_sem)
            s1.wait()
            s2.wait()

        plsc.subcore_barrier()
        t1 = pltpu.async_copy(us_sh, us_v, tab_sem)
        t2 = pltpu.async_copy(is_sh, is_v, tab_sem)
        in_copies = []
        for k in range(_N_CHUNKS):
            o = k * ce
            in_copies.append((
                pltpu.async_copy(src_hbm.at[pl.ds(base + o, ce)],
                                 src_v.at[pl.ds(o, ce)], in_sems[k]),
                pltpu.async_copy(dst_hbm.at[pl.ds(base + o, ce)],
                                 dst_v.at[pl.ds(o, ce)], in_sems[k]),
            ))
        t1.wait()
        t2.wait()
        out_copies = []
        for k in range(_N_CHUNKS):
            c1, c2 = in_copies[k]
            c1.wait()
            c2.wait()

            @plsc.parallel_loop(k * citer, (k + 1) * citer, unroll=5)
            def body(i):
                sl = pl.ds(i * 16, 16)
                vu = plsc.load_gather(us_v, [src_v[sl]])
                vi = plsc.load_gather(is_v, [dst_v[sl]])
                out_v[sl] = vu + vi

            o = k * ce
            out_copies.append(
                pltpu.async_copy(out_v.at[pl.ds(o, ce)],
                                 out_hbm.at[pl.ds(base + o, ce)], out_sem))
        for c in out_copies:
            c.wait()

    return sc_kernel(u_score, i_score, edge_src, edge_dst)


def kernel(user_feat, item_feat, edge_src, edge_dst, W, b):
    u_score, i_score = _node_scores(user_feat, item_feat, W, b)
    out = _edge_scores(u_score, i_score, edge_src, edge_dst)
    return out.reshape(edge_src.shape[0], 1)


# speedup vs baseline: 1.1028x; 1.0003x over previous
import functools

import jax
import jax.numpy as jnp
from jax import lax
from jax.experimental import pallas as pl
from jax.experimental.pallas import tpu as pltpu
from jax.experimental.pallas import tpu_sc as plsc


def _node_scores_body(block, u_ref, i_ref, w_ref, b_ref, us_ref, is_ref):
    g = pl.program_id(0)
    d_u = u_ref.shape[1]
    wu = w_ref[:, :d_u]
    wi = w_ref[:, d_u:]
    dn = (((1,), (1,)), ((), ()))
    us = lax.dot_general(wu, u_ref[...], dn, preferred_element_type=jnp.float32)
    vs = lax.dot_general(wi, i_ref[...], dn, preferred_element_type=jnp.float32)
    sl = pl.ds(g * block, block)
    us_ref[sl] = us[0]
    is_ref[sl] = vs[0] + b_ref[0, 0]


def _node_scores(user_feat, item_feat, W, b):
    n_user, d_u = user_feat.shape
    n_item, d_i = item_feat.shape
    block = 2048
    grid = ((n_user + block - 1) // block,)
    n_pad = grid[0] * block
    b2 = b.reshape(1, 1)
    return pl.pallas_call(
        functools.partial(_node_scores_body, block),
        grid=grid,
        in_specs=[
            pl.BlockSpec((block, d_u), lambda i: (i, 0)),
            pl.BlockSpec((block, d_i), lambda i: (i, 0)),
            pl.BlockSpec((1, d_u + d_i), lambda i: (0, 0)),
            pl.BlockSpec((1, 1), lambda i: (0, 0)),
        ],
        out_specs=[
            pl.BlockSpec((n_pad,), lambda i: (0,)),
            pl.BlockSpec((n_pad,), lambda i: (0,)),
        ],
        out_shape=[
            jax.ShapeDtypeStruct((n_pad,), jnp.float32),
            jax.ShapeDtypeStruct((n_pad,), jnp.float32),
        ],
    )(user_feat, item_feat, W, b2)


_N_CHUNKS = 5


def _edge_scores(u_score, i_score, edge_src, edge_dst):
    n_user = u_score.shape[0]
    n_item = i_score.shape[0]
    e = edge_src.shape[0]
    info = plsc.get_sparse_core_info()
    nc, ns = info.num_cores, info.num_subcores
    nw = nc * ns
    epw = e // nw
    mesh = plsc.VectorSubcoreMesh(core_axis_name="c", subcore_axis_name="s")

    @functools.partial(
        pl.kernel,
        mesh=mesh,
        compiler_params=pltpu.CompilerParams(
            needs_layout_passes=False, use_tc_tiling_on_sc=False),
        out_type=jax.ShapeDtypeStruct((e,), jnp.float32),
        scratch_types=[
            pltpu.VMEM((epw,), jnp.int32),
            pltpu.VMEM((epw,), jnp.int32),
            pltpu.VMEM((n_user,), jnp.float32),
            pltpu.VMEM((n_item,), jnp.float32),
            pltpu.VMEM((epw,), jnp.float32),
            pltpu.VMEM_SHARED((n_user,), jnp.float32),
            pltpu.VMEM_SHARED((n_item,), jnp.float32),
            pltpu.SemaphoreType.DMA,
            pltpu.SemaphoreType.DMA,
            [pltpu.SemaphoreType.DMA] * _N_CHUNKS,
        ],
    )
    def sc_kernel(us_hbm, is_hbm, src_hbm, dst_hbm, out_hbm,
                  src_v, dst_v, us_v, is_v, out_v, us_sh, is_sh,
                  tab_sem, out_sem, in_sems):
        sid = lax.axis_index("s")
        wid = sid * nc + lax.axis_index("c")
        base = wid * epw
        ce = epw // _N_CHUNKS
        citer = ce // 16

        @pl.when(sid == 0)
        def _():
            s1 = pltpu.async_copy(us_hbm, us_sh, tab_sem)
            s2 = pltpu.async_copy(is_hbm, is_sh, tab_sem)
            s1.wait()
            s2.wait()

        plsc.subcore_barrier()
        t1 = pltpu.async_copy(us_sh, us_v, tab_sem)
        t2 = pltpu.async_copy(is_sh, is_v, tab_sem)
        in_copies = []
        for k in range(_N_CHUNKS):
            o = k * ce
            in_copies.append((
                pltpu.async_copy(src_hbm.at[pl.ds(base + o, ce)],
                                 src_v.at[pl.ds(o, ce)], in_sems[k]),
                pltpu.async_copy(dst_hbm.at[pl.ds(base + o, ce)],
                                 dst_v.at[pl.ds(o, ce)], in_sems[k]),
            ))
        t1.wait()
        t2.wait()
        out_copies = []
        for k in range(_N_CHUNKS):
            c1, c2 = in_copies[k]
            c1.wait()
            c2.wait()

            @plsc.parallel_loop(k * citer, (k + 1) * citer, unroll=5)
            def body(i):
                sl = pl.ds(i * 16, 16)
                vu = plsc.load_gather(us_v, [src_v[sl]])
                vi = plsc.load_gather(is_v, [dst_v[sl]])
                out_v[sl] = vu + vi

            o = k * ce
            out_copies.append(
                pltpu.async_copy(out_v.at[pl.ds(o, ce)],
                                 out_hbm.at[pl.ds(base + o, ce)], out_sem))
        for c in out_copies:
            c.wait()

    return sc_kernel(u_score, i_score, edge_src, edge_dst)


def kernel(user_feat, item_feat, edge_src, edge_dst, W, b):
    u_score, i_score = _node_scores(user_feat, item_feat, W, b)
    out = _edge_scores(u_score, i_score, edge_src, edge_dst)
    return lax.broadcast_in_dim(out, (edge_src.shape[0], 1), (0,))
